# baseline (speedup 1.0000x reference)
import functools

import jax
import jax.numpy as jnp
from jax import lax
from jax.experimental import pallas as pl
from jax.experimental.pallas import tpu as pltpu
from jax.experimental.pallas import tpu_sc as plsc

N_NODES = 10000
N_EDGES = 320000
NC = 2
NS = 16
NW = NC * NS
EPW = N_EDGES // NW
SUB = 80
NSUB = EPW // SUB
NPAD = 10240
ROWS_PT = NPAD // NS


def _tanh(z):
    e = jnp.exp(2.0 * z)
    return 1.0 - 2.0 / (e + 1.0)


def _sigmoid(z):
    return 1.0 / (1.0 + jnp.exp(-z))


def _f32(shape):
    return jax.ShapeDtypeStruct(shape, jnp.float32)


_MESH = plsc.VectorSubcoreMesh(core_axis_name="c", subcore_axis_name="s")


def _lane(v, j):
    return lax.gather(
        v, jnp.full((16, 1), j, jnp.int32),
        lax.GatherDimensionNumbers(offset_dims=(), collapsed_slice_dims=(0,),
                                   start_index_map=(0,)),
        (1,), mode=lax.GatherScatterMode.PROMISE_IN_BOUNDS)


def _edge_common(rows1, rows2, b0v, w1v, iota16, g):
    ridx = g * 16 + iota16
    acc = _lane(b0v, 8)
    for j in range(8):
        a = plsc.load_gather(rows1, [ridx, jnp.full((16,), 8 + j, jnp.int32)])
        b = plsc.load_gather(rows2, [ridx, jnp.full((16,), j, jnp.int32)])
        acc = acc + _lane(w1v, j) * _tanh(a + b + _lane(b0v, j))
    return ridx, _sigmoid(acc)


def _sc_edge_body(t1, t2, st, en, wp, zr, m1_out, m2_out,
                  sidx, eidx, rows1a, rows2a, rows1b, rows2b, msga, msgb, wv,
                  sem1a, sem2a, sem1b, sem2b, semma, semmb, m1_sp, m2_sp):
    cid = lax.axis_index("c")
    sid = lax.axis_index("s")
    wid = sid * NC + cid
    rbase = sid * ROWS_PT
    pltpu.sync_copy(zr.at[pl.ds(rbase, ROWS_PT)], m1_sp.at[pl.ds(rbase, ROWS_PT)])
    pltpu.sync_copy(zr.at[pl.ds(rbase, ROWS_PT)], m2_sp.at[pl.ds(rbase, ROWS_PT)])
    pltpu.sync_copy(wp, wv)
    pltpu.sync_copy(st.at[wid], sidx)
    pltpu.sync_copy(en.at[wid], eidx)
    plsc.subcore_barrier()
    iota16 = lax.iota(jnp.int32, 16)
    b0v = wv[0:16]
    w1v = wv[16:32]

    gbufs = ((rows1a, rows2a, sem1a, sem2a), (rows1b, rows2b, sem1b, sem2b))
    mbufs = ((msga, semma), (msgb, semmb))

    def compute(k, rows1, rows2, msg, semm):
        for g in range(SUB // 16):
            ridx, s = _edge_common(rows1, rows2, b0v, w1v, iota16, g)
            for j in range(8):
                p = plsc.load_gather(rows1, [ridx, jnp.full((16,), j, jnp.int32)])
                q = plsc.load_gather(rows2, [ridx, jnp.full((16,), 8 + j, jnp.int32)])
                plsc.store_scatter(msg, [ridx, jnp.full((16,), j, jnp.int32)], s * p)
                plsc.store_scatter(msg, [ridx, jnp.full((16,), 8 + j, jnp.int32)], s * q)
        pltpu.async_copy(msg, m1_sp.at[eidx.at[k]], semm, add=True)
        pltpu.async_copy(msg, m2_sp.at[sidx.at[k]], semm, add=True)

    def drain_msg(k, msg, semm):
        pltpu.make_async_copy(msg, m1_sp.at[eidx.at[k]], semm).wait()
        pltpu.make_async_copy(msg, m2_sp.at[sidx.at[k]], semm).wait()

    def step(k, b, do_drain, do_issue):
        r1, r2, s1, s2 = gbufs[b]
        n1, n2, t1s, t2s = gbufs[1 - b]
        msg, semm = mbufs[b]
        pltpu.make_async_copy(t1.at[sidx.at[k]], r1, s1).wait()
        pltpu.make_async_copy(t2.at[eidx.at[k]], r2, s2).wait()
        if do_issue:
            pltpu.async_copy(t1.at[sidx.at[k + 1]], n1, t1s)
            pltpu.async_copy(t2.at[eidx.at[k + 1]], n2, t2s)
        if do_drain:
            drain_msg(k, msg, semm)
        compute(k, r1, r2, msg, semm)

    pltpu.async_copy(t1.at[sidx.at[0]], rows1a, sem1a)
    pltpu.async_copy(t2.at[eidx.at[0]], rows2a, sem2a)
    step(0, 0, False, True)
    step(1, 1, False, True)

    def pair(i, carry):
        step(2 * i, 0, True, True)
        step(2 * i + 1, 1, True, True)
        return carry

    lax.fori_loop(1, (NSUB - 1) // 2, pair, 0)
    step(NSUB - 1, 0, True, False)
    drain_msg(NSUB - 2, msgb, semmb)
    drain_msg(NSUB - 1, msga, semma)
    plsc.subcore_barrier()
    pltpu.sync_copy(m1_sp.at[pl.ds(rbase, ROWS_PT)],
                    m1_out.at[cid, pl.ds(rbase, ROWS_PT)])
    pltpu.sync_copy(m2_sp.at[pl.ds(rbase, ROWS_PT)],
                    m2_out.at[cid, pl.ds(rbase, ROWS_PT)])


_sc_edge = pl.kernel(
    _sc_edge_body,
    out_type=(_f32((NC, NPAD, 16)), _f32((NC, NPAD, 16))),
    mesh=_MESH,
    scratch_types=[
        pltpu.VMEM((NSUB, SUB), jnp.int32),
        pltpu.VMEM((NSUB, SUB), jnp.int32),
        pltpu.VMEM((SUB, 16), jnp.float32),
        pltpu.VMEM((SUB, 16), jnp.float32),
        pltpu.VMEM((SUB, 16), jnp.float32),
        pltpu.VMEM((SUB, 16), jnp.float32),
        pltpu.VMEM((SUB, 16), jnp.float32),
        pltpu.VMEM((SUB, 16), jnp.float32),
        pltpu.VMEM((32,), jnp.float32),
        pltpu.SemaphoreType.DMA,
        pltpu.SemaphoreType.DMA,
        pltpu.SemaphoreType.DMA,
        pltpu.SemaphoreType.DMA,
        pltpu.SemaphoreType.DMA,
        pltpu.SemaphoreType.DMA,
        pltpu.VMEM_SHARED((NPAD, 16), jnp.float32),
        pltpu.VMEM_SHARED((NPAD, 16), jnp.float32),
    ],
    compiler_params=pltpu.CompilerParams(needs_layout_passes=False, use_tc_tiling_on_sc=False),
    name="sc_edge_pass",
)


def _sc_final_body(t1, t2, st, en, wp, s_out,
                   sidx, eidx, rows1a, rows2a, rows1b, rows2b, sbuf, wv,
                   sem1a, sem2a, sem1b, sem2b):
    cid = lax.axis_index("c")
    sid = lax.axis_index("s")
    wid = sid * NC + cid
    pltpu.sync_copy(wp, wv)
    pltpu.sync_copy(st.at[wid], sidx)
    pltpu.sync_copy(en.at[wid], eidx)
    iota16 = lax.iota(jnp.int32, 16)
    b0v = wv[0:16]
    w1v = wv[16:32]

    def compute(k, rows1, rows2):
        for g in range(SUB // 16):
            _, s = _edge_common(rows1, rows2, b0v, w1v, iota16, g)
            sbuf[pl.ds(k * SUB + g * 16, 16)] = s

    bufs = ((rows1a, rows2a, sem1a, sem2a), (rows1b, rows2b, sem1b, sem2b))
    pltpu.async_copy(t1.at[sidx.at[0]], rows1a, sem1a)
    pltpu.async_copy(t2.at[eidx.at[0]], rows2a, sem2a)

    def pair(i, carry):
        for b in range(2):
            k = 2 * i + b
            r1, r2, s1, s2 = bufs[b]
            n1, n2, t1s, t2s = bufs[1 - b]
            pltpu.make_async_copy(t1.at[sidx.at[k]], r1, s1).wait()
            pltpu.make_async_copy(t2.at[eidx.at[k]], r2, s2).wait()
            pltpu.async_copy(t1.at[sidx.at[k + 1]], n1, t1s)
            pltpu.async_copy(t2.at[eidx.at[k + 1]], n2, t2s)
            compute(k, r1, r2)
        return carry

    lax.fori_loop(0, (NSUB - 1) // 2, pair, 0)
    k = NSUB - 1
    pltpu.make_async_copy(t1.at[sidx.at[k]], rows1a, sem1a).wait()
    pltpu.make_async_copy(t2.at[eidx.at[k]], rows2a, sem2a).wait()
    compute(k, rows1a, rows2a)
    pltpu.sync_copy(sbuf, s_out.at[wid])


_sc_final = pl.kernel(
    _sc_final_body,
    out_type=_f32((NW, EPW)),
    mesh=_MESH,
    scratch_types=[
        pltpu.VMEM((NSUB, SUB), jnp.int32),
        pltpu.VMEM((NSUB, SUB), jnp.int32),
        pltpu.VMEM((SUB, 16), jnp.float32),
        pltpu.VMEM((SUB, 16), jnp.float32),
        pltpu.VMEM((SUB, 16), jnp.float32),
        pltpu.VMEM((SUB, 16), jnp.float32),
        pltpu.VMEM((EPW,), jnp.float32),
        pltpu.VMEM((32,), jnp.float32),
        pltpu.SemaphoreType.DMA,
        pltpu.SemaphoreType.DMA,
        pltpu.SemaphoreType.DMA,
        pltpu.SemaphoreType.DMA,
    ],
    compiler_params=pltpu.CompilerParams(needs_layout_passes=False, use_tc_tiling_on_sc=False),
    name="sc_edge_final",
)


def _ln(z, g, b, eps=1e-5):
    mu = jnp.mean(z, axis=-1, keepdims=True)
    var = jnp.mean((z - mu) ** 2, axis=-1, keepdims=True)
    return (z - mu) / jnp.sqrt(var + eps) * g + b


def _split_tall(tall, t1_ref, t2_ref, r_ref):
    t1_ref[...] = tall[:, 0:16]
    t2_ref[...] = tall[:, 16:32]
    r_ref[...] = tall[:, 32:40]


def _tc_d0_body(x_ref, w0t_ref, b0_ref, g0_ref, be0_ref, w1t_ref, b1_ref,
                g1_ref, be1_ref, w2t_ref, b2_ref, g2_ref, be2_ref,
                w8_ref, wxt_ref, bias40_ref,
                h_ref, xp_ref, t1_ref, t2_ref, r_ref):
    x = x_ref[...]
    h = jnp.tanh(_ln(x @ w0t_ref[...] + b0_ref[...], g0_ref[...], be0_ref[...]))
    h = jnp.tanh(_ln(h @ w1t_ref[...] + b1_ref[...], g1_ref[...], be1_ref[...]))
    h = jnp.tanh(_ln(h @ w2t_ref[...] + b2_ref[...], g2_ref[...], be2_ref[...]))
    h_ref[...] = h
    xp = x @ wxt_ref[...] + bias40_ref[...]
    xp_ref[...] = xp
    _split_tall((2.0 * h) @ w8_ref[...] + xp, t1_ref, t2_ref, r_ref)


def _tc_d2_body(m1_ref, m2_ref, r_ref, h_ref, xp_ref, w1t_ref, b1_ref, w8_ref,
                hn_ref, t1_ref, t2_ref, rn_ref):
    m = (m1_ref[0, :, 0:8] + m1_ref[1, :, 0:8]
         + m2_ref[0, :, 8:16] + m2_ref[1, :, 8:16])
    t = jnp.tanh(m + r_ref[...])
    hn = jnp.tanh(t @ w1t_ref[...] + b1_ref[...])
    hn_ref[...] = hn
    c8 = h_ref[...] + hn
    _split_tall(c8 @ w8_ref[...] + xp_ref[...], t1_ref, t2_ref, rn_ref)


_BLK = 1000
_NBLK = N_NODES // _BLK


def _full(shape):
    return pl.BlockSpec(shape, lambda i: (0,) * len(shape))


def _rows(w):
    return pl.BlockSpec((_BLK, w), lambda i: (i, 0))


_D0_OUT = (_f32((N_NODES, 8)), _f32((N_NODES, 40)), _f32((N_NODES, 16)),
           _f32((N_NODES, 16)), _f32((N_NODES, 8)))
_tc_d0 = pl.pallas_call(
    _tc_d0_body,
    grid=(_NBLK,),
    in_specs=[_rows(128)] + [_full(s) for s in
              [(128, 8), (1, 8), (1, 8), (1, 8), (8, 8), (1, 8), (1, 8), (1, 8),
               (8, 8), (1, 8), (1, 8), (1, 8), (8, 40), (128, 40), (1, 40)]],
    out_specs=(_rows(8), _rows(40), _rows(16), _rows(16), _rows(8)),
    out_shape=_D0_OUT,
)

_D2_OUT = (_f32((N_NODES, 8)), _f32((N_NODES, 16)), _f32((N_NODES, 16)),
           _f32((N_NODES, 8)))
_tc_d2 = pl.pallas_call(
    _tc_d2_body,
    grid=(_NBLK,),
    in_specs=[pl.BlockSpec((NC, _BLK, 16), lambda i: (0, i, 0)),
              pl.BlockSpec((NC, _BLK, 16), lambda i: (0, i, 0)),
              _rows(8), _rows(8), _rows(40),
              _full((8, 8)), _full((1, 8)), _full((8, 40))],
    out_specs=(_rows(8), _rows(16), _rows(16), _rows(8)),
    out_shape=_D2_OUT,
)


@functools.partial(jax.jit, static_argnums=())
def kernel(nodes, edges, in_W0, in_b0, in_g0, in_be0, in_W1, in_b1, in_g1,
           in_be1, in_W2, in_b2, in_g2, in_be2, e_W0, e_b0, e_W1, e_b1,
           n_W0, n_b0, n_W1, n_b1):
    x = nodes[0]
    ei = edges[0].astype(jnp.int32)
    start_r = ei[:, 0].reshape(NW, NSUB, SUB)
    end_r = ei[:, 1].reshape(NW, NSUB, SUB)

    c = 136
    wbig = jnp.concatenate([n_W0[:, :c], e_W0[:, :c], e_W0[:, c:],
                            n_W0[:, c:2 * c], n_W0[:, 2 * c:]], axis=0).T
    w8 = wbig[:8]
    wxt = wbig[8:]
    bias40 = jnp.concatenate([jnp.zeros((32,), jnp.float32), n_b0])[None, :]

    wp = jnp.zeros((32,), jnp.float32)
    wp = wp.at[0:8].set(e_b0).at[8].set(e_b1[0]).at[16:24].set(e_W1[0])
    zr = jnp.zeros((NPAD, 16), jnp.float32)

    h, xp, t1, t2, r = _tc_d0(
        x, in_W0.T, in_b0[None, :], in_g0[None, :], in_be0[None, :],
        in_W1.T, in_b1[None, :], in_g1[None, :], in_be1[None, :],
        in_W2.T, in_b2[None, :], in_g2[None, :], in_be2[None, :],
        w8, wxt, bias40)

    for _ in range(3):
        m1, m2 = _sc_edge(t1, t2, start_r, end_r, wp, zr)
        h, t1, t2, r = _tc_d2(m1, m2, r, h, xp, n_W1.T, n_b1[None, :], w8)

    s = _sc_final(t1, t2, start_r, end_r, wp)
    return s.reshape(1, N_EDGES)


# speedup vs baseline: 1.0486x; 1.0486x over previous
import functools

import jax
import jax.numpy as jnp
from jax import lax
from jax.experimental import pallas as pl
from jax.experimental.pallas import tpu as pltpu
from jax.experimental.pallas import tpu_sc as plsc

N_NODES = 10000
N_EDGES = 320000
NC = 2
NS = 16
NW = NC * NS
EPW = N_EDGES // NW
SUB = 80
NSUB = EPW // SUB
NPAD = 10240
ROWS_PT = NPAD // NS


def _sigmoid(z):
    return 1.0 / (1.0 + jnp.exp(-z))


def _f32(shape):
    return jax.ShapeDtypeStruct(shape, jnp.float32)


_MESH = plsc.VectorSubcoreMesh(core_axis_name="c", subcore_axis_name="s")


def _lane(v, j):
    return lax.gather(
        v, jnp.full((16, 1), j, jnp.int32),
        lax.GatherDimensionNumbers(offset_dims=(), collapsed_slice_dims=(0,),
                                   start_index_map=(0,)),
        (1,), mode=lax.GatherScatterMode.PROMISE_IN_BOUNDS)


def _edge_common(rows1, rows2, acc0, w2, iota16, g):
    ridx = g * 16 + iota16
    acc = acc0
    for j in range(8):
        a = plsc.load_gather(rows1, [ridx, jnp.full((16,), 8 + j, jnp.int32)])
        b = plsc.load_gather(rows2, [ridx, jnp.full((16,), j, jnp.int32)])
        acc = acc - w2[j] / (jnp.exp(a + b) + 1.0)
    return ridx, _sigmoid(acc)


def _sc_edge_body(t1, t2, st, en, wp, zr, m1_out, m2_out,
                  sidx, eidx, rows1a, rows2a, rows1b, rows2b, msg, wv,
                  sem1a, sem2a, sem1b, sem2b, m1_sp, m2_sp):
    cid = lax.axis_index("c")
    sid = lax.axis_index("s")
    wid = sid * NC + cid
    rbase = sid * ROWS_PT
    pltpu.sync_copy(zr.at[pl.ds(rbase, ROWS_PT)], m1_sp.at[pl.ds(rbase, ROWS_PT)])
    pltpu.sync_copy(zr.at[pl.ds(rbase, ROWS_PT)], m2_sp.at[pl.ds(rbase, ROWS_PT)])
    pltpu.sync_copy(wp, wv)
    pltpu.sync_copy(st.at[wid], sidx)
    pltpu.sync_copy(en.at[wid], eidx)
    plsc.subcore_barrier()
    iota16 = lax.iota(jnp.int32, 16)
    wv16 = wv[0:16]
    acc0 = _lane(wv16, 8)
    w2 = [_lane(wv16, j) for j in range(8)]

    gbufs = ((rows1a, rows2a, sem1a, sem2a), (rows1b, rows2b, sem1b, sem2b))

    def compute(k, rows1, rows2):
        for g in range(SUB // 16):
            ridx, s = _edge_common(rows1, rows2, acc0, w2, iota16, g)
            for j in range(8):
                p = plsc.load_gather(rows1, [ridx, jnp.full((16,), j, jnp.int32)])
                q = plsc.load_gather(rows2, [ridx, jnp.full((16,), 8 + j, jnp.int32)])
                plsc.store_scatter(msg, [ridx, jnp.full((16,), j, jnp.int32)], s * p)
                plsc.store_scatter(msg, [ridx, jnp.full((16,), 8 + j, jnp.int32)], s * q)
        pltpu.sync_copy(msg, m1_sp.at[eidx.at[k]], add=True)
        pltpu.sync_copy(msg, m2_sp.at[sidx.at[k]], add=True)

    def step(k, b, do_issue):
        r1, r2, s1, s2 = gbufs[b]
        n1, n2, t1s, t2s = gbufs[1 - b]
        pltpu.make_async_copy(t1.at[sidx.at[k]], r1, s1).wait()
        pltpu.make_async_copy(t2.at[eidx.at[k]], r2, s2).wait()
        if do_issue:
            pltpu.async_copy(t1.at[sidx.at[k + 1]], n1, t1s)
            pltpu.async_copy(t2.at[eidx.at[k + 1]], n2, t2s)
        compute(k, r1, r2)

    pltpu.async_copy(t1.at[sidx.at[0]], rows1a, sem1a)
    pltpu.async_copy(t2.at[eidx.at[0]], rows2a, sem2a)

    def pair(i, carry):
        step(2 * i, 0, True)
        step(2 * i + 1, 1, True)
        return carry

    lax.fori_loop(0, (NSUB - 1) // 2, pair, 0)
    step(NSUB - 1, 0, False)
    plsc.subcore_barrier()
    pltpu.sync_copy(m1_sp.at[pl.ds(rbase, ROWS_PT)],
                    m1_out.at[cid, pl.ds(rbase, ROWS_PT)])
    pltpu.sync_copy(m2_sp.at[pl.ds(rbase, ROWS_PT)],
                    m2_out.at[cid, pl.ds(rbase, ROWS_PT)])


_sc_edge = pl.kernel(
    _sc_edge_body,
    out_type=(_f32((NC, NPAD, 16)), _f32((NC, NPAD, 16))),
    mesh=_MESH,
    scratch_types=[
        pltpu.VMEM((NSUB, SUB), jnp.int32),
        pltpu.VMEM((NSUB, SUB), jnp.int32),
        pltpu.VMEM((SUB, 16), jnp.float32),
        pltpu.VMEM((SUB, 16), jnp.float32),
        pltpu.VMEM((SUB, 16), jnp.float32),
        pltpu.VMEM((SUB, 16), jnp.float32),
        pltpu.VMEM((SUB, 16), jnp.float32),
        pltpu.VMEM((32,), jnp.float32),
        pltpu.SemaphoreType.DMA,
        pltpu.SemaphoreType.DMA,
        pltpu.SemaphoreType.DMA,
        pltpu.SemaphoreType.DMA,
        pltpu.VMEM_SHARED((NPAD, 16), jnp.float32),
        pltpu.VMEM_SHARED((NPAD, 16), jnp.float32),
    ],
    compiler_params=pltpu.CompilerParams(needs_layout_passes=False, use_tc_tiling_on_sc=False),
    name="sc_edge_pass",
)


def _sc_final_body(t1, t2, st, en, wp, s_out,
                   sidx, eidx, rows1a, rows2a, rows1b, rows2b, sbuf, wv,
                   sem1a, sem2a, sem1b, sem2b):
    cid = lax.axis_index("c")
    sid = lax.axis_index("s")
    wid = sid * NC + cid
    pltpu.sync_copy(wp, wv)
    pltpu.sync_copy(st.at[wid], sidx)
    pltpu.sync_copy(en.at[wid], eidx)
    iota16 = lax.iota(jnp.int32, 16)
    wv16 = wv[0:16]
    acc0 = _lane(wv16, 8)
    w2 = [_lane(wv16, j) for j in range(8)]

    def compute(k, rows1, rows2):
        for g in range(SUB // 16):
            _, s = _edge_common(rows1, rows2, acc0, w2, iota16, g)
            sbuf[pl.ds(k * SUB + g * 16, 16)] = s

    bufs = ((rows1a, rows2a, sem1a, sem2a), (rows1b, rows2b, sem1b, sem2b))
    pltpu.async_copy(t1.at[sidx.at[0]], rows1a, sem1a)
    pltpu.async_copy(t2.at[eidx.at[0]], rows2a, sem2a)

    def pair(i, carry):
        for b in range(2):
            k = 2 * i + b
            r1, r2, s1, s2 = bufs[b]
            n1, n2, t1s, t2s = bufs[1 - b]
            pltpu.make_async_copy(t1.at[sidx.at[k]], r1, s1).wait()
            pltpu.make_async_copy(t2.at[eidx.at[k]], r2, s2).wait()
            pltpu.async_copy(t1.at[sidx.at[k + 1]], n1, t1s)
            pltpu.async_copy(t2.at[eidx.at[k + 1]], n2, t2s)
            compute(k, r1, r2)
        return carry

    lax.fori_loop(0, (NSUB - 1) // 2, pair, 0)
    k = NSUB - 1
    pltpu.make_async_copy(t1.at[sidx.at[k]], rows1a, sem1a).wait()
    pltpu.make_async_copy(t2.at[eidx.at[k]], rows2a, sem2a).wait()
    compute(k, rows1a, rows2a)
    pltpu.sync_copy(sbuf, s_out.at[wid])


_sc_final = pl.kernel(
    _sc_final_body,
    out_type=_f32((NW, EPW)),
    mesh=_MESH,
    scratch_types=[
        pltpu.VMEM((NSUB, SUB), jnp.int32),
        pltpu.VMEM((NSUB, SUB), jnp.int32),
        pltpu.VMEM((SUB, 16), jnp.float32),
        pltpu.VMEM((SUB, 16), jnp.float32),
        pltpu.VMEM((SUB, 16), jnp.float32),
        pltpu.VMEM((SUB, 16), jnp.float32),
        pltpu.VMEM((EPW,), jnp.float32),
        pltpu.VMEM((32,), jnp.float32),
        pltpu.SemaphoreType.DMA,
        pltpu.SemaphoreType.DMA,
        pltpu.SemaphoreType.DMA,
        pltpu.SemaphoreType.DMA,
    ],
    compiler_params=pltpu.CompilerParams(needs_layout_passes=False, use_tc_tiling_on_sc=False),
    name="sc_edge_final",
)


def _ln(z, g, b, eps=1e-5):
    mu = jnp.mean(z, axis=-1, keepdims=True)
    var = jnp.mean((z - mu) ** 2, axis=-1, keepdims=True)
    return (z - mu) / jnp.sqrt(var + eps) * g + b


def _split_tall(tall, t1_ref, t2_ref, r_ref):
    t1_ref[...] = tall[:, 0:16]
    t2_ref[...] = tall[:, 16:32]
    r_ref[...] = tall[:, 32:40]


def _tc_d0_body(x_ref, w0t_ref, b0_ref, g0_ref, be0_ref, w1t_ref, b1_ref,
                g1_ref, be1_ref, w2t_ref, b2_ref, g2_ref, be2_ref,
                w8_ref, wxt_ref, bias40_ref,
                h_ref, xp_ref, t1_ref, t2_ref, r_ref):
    x = x_ref[...]
    h = jnp.tanh(_ln(x @ w0t_ref[...] + b0_ref[...], g0_ref[...], be0_ref[...]))
    h = jnp.tanh(_ln(h @ w1t_ref[...] + b1_ref[...], g1_ref[...], be1_ref[...]))
    h = jnp.tanh(_ln(h @ w2t_ref[...] + b2_ref[...], g2_ref[...], be2_ref[...]))
    h_ref[...] = h
    xp = x @ wxt_ref[...] + bias40_ref[...]
    xp_ref[...] = xp
    _split_tall((2.0 * h) @ w8_ref[...] + xp, t1_ref, t2_ref, r_ref)


def _tc_d2_body(m1_ref, m2_ref, r_ref, h_ref, xp_ref, w1t_ref, b1_ref, w8_ref,
                hn_ref, t1_ref, t2_ref, rn_ref):
    m = (m1_ref[0, :, 0:8] + m1_ref[1, :, 0:8]
         + m2_ref[0, :, 8:16] + m2_ref[1, :, 8:16])
    t = jnp.tanh(m + r_ref[...])
    hn = jnp.tanh(t @ w1t_ref[...] + b1_ref[...])
    hn_ref[...] = hn
    c8 = h_ref[...] + hn
    _split_tall(c8 @ w8_ref[...] + xp_ref[...], t1_ref, t2_ref, rn_ref)


_BLK = 1000
_NBLK = N_NODES // _BLK


def _full(shape):
    return pl.BlockSpec(shape, lambda i: (0,) * len(shape))


def _rows(w):
    return pl.BlockSpec((_BLK, w), lambda i: (i, 0))


_D0_OUT = (_f32((N_NODES, 8)), _f32((N_NODES, 40)), _f32((N_NODES, 16)),
           _f32((N_NODES, 16)), _f32((N_NODES, 8)))
_tc_d0 = pl.pallas_call(
    _tc_d0_body,
    grid=(_NBLK,),
    in_specs=[_rows(128)] + [_full(s) for s in
              [(128, 8), (1, 8), (1, 8), (1, 8), (8, 8), (1, 8), (1, 8), (1, 8),
               (8, 8), (1, 8), (1, 8), (1, 8), (8, 40), (128, 40), (1, 40)]],
    out_specs=(_rows(8), _rows(40), _rows(16), _rows(16), _rows(8)),
    out_shape=_D0_OUT,
)

_D2_OUT = (_f32((N_NODES, 8)), _f32((N_NODES, 16)), _f32((N_NODES, 16)),
           _f32((N_NODES, 8)))
_tc_d2 = pl.pallas_call(
    _tc_d2_body,
    grid=(_NBLK,),
    in_specs=[pl.BlockSpec((NC, _BLK, 16), lambda i: (0, i, 0)),
              pl.BlockSpec((NC, _BLK, 16), lambda i: (0, i, 0)),
              _rows(8), _rows(8), _rows(40),
              _full((8, 8)), _full((1, 8)), _full((8, 40))],
    out_specs=(_rows(8), _rows(16), _rows(16), _rows(8)),
    out_shape=_D2_OUT,
)


@functools.partial(jax.jit, static_argnums=())
def kernel(nodes, edges, in_W0, in_b0, in_g0, in_be0, in_W1, in_b1, in_g1,
           in_be1, in_W2, in_b2, in_g2, in_be2, e_W0, e_b0, e_W1, e_b1,
           n_W0, n_b0, n_W1, n_b1):
    x = nodes[0]
    ei = edges[0].astype(jnp.int32)
    start_r = ei[:, 0].reshape(NW, NSUB, SUB)
    end_r = ei[:, 1].reshape(NW, NSUB, SUB)

    c = 136
    wbig = jnp.concatenate([n_W0[:, :c], e_W0[:, :c], e_W0[:, c:],
                            n_W0[:, c:2 * c], n_W0[:, 2 * c:]], axis=0).T
    colscale = jnp.concatenate([jnp.ones((8,), jnp.float32),
                                jnp.full((16,), 2.0, jnp.float32),
                                jnp.ones((16,), jnp.float32)])
    wbig = wbig * colscale[None, :]
    w8 = wbig[:8]
    wxt = wbig[8:]
    bias40 = jnp.concatenate([jnp.zeros((8,), jnp.float32), 2.0 * e_b0,
                              jnp.zeros((16,), jnp.float32), n_b0])[None, :]

    wp = jnp.zeros((32,), jnp.float32)
    wp = wp.at[0:8].set(2.0 * e_W1[0]).at[8].set(e_b1[0] + jnp.sum(e_W1[0]))
    zr = jnp.zeros((NPAD, 16), jnp.float32)

    h, xp, t1, t2, r = _tc_d0(
        x, in_W0.T, in_b0[None, :], in_g0[None, :], in_be0[None, :],
        in_W1.T, in_b1[None, :], in_g1[None, :], in_be1[None, :],
        in_W2.T, in_b2[None, :], in_g2[None, :], in_be2[None, :],
        w8, wxt, bias40)

    for _ in range(3):
        m1, m2 = _sc_edge(t1, t2, start_r, end_r, wp, zr)
        h, t1, t2, r = _tc_d2(m1, m2, r, h, xp, n_W1.T, n_b1[None, :], w8)

    s = _sc_final(t1, t2, start_r, end_r, wp)
    return s.reshape(1, N_EDGES)


# speedup vs baseline: 1.2182x; 1.1618x over previous
import functools

import jax
import jax.numpy as jnp
from jax import lax
from jax.experimental import pallas as pl
from jax.experimental.pallas import tpu as pltpu
from jax.experimental.pallas import tpu_sc as plsc

N_NODES = 10000
N_EDGES = 320000
NC = 2
NS = 16
NW = NC * NS
EPW = N_EDGES // NW
SUB = 80
NSUB = EPW // SUB
NPAD = 10240
ROWS_PT = NPAD // NS


def _sigmoid(z):
    return 1.0 / (1.0 + jnp.exp(-z))


def _f32(shape):
    return jax.ShapeDtypeStruct(shape, jnp.float32)


_MESH = plsc.VectorSubcoreMesh(core_axis_name="c", subcore_axis_name="s")


def _lane(v, j):
    return lax.gather(
        v, jnp.full((16, 1), j, jnp.int32),
        lax.GatherDimensionNumbers(offset_dims=(), collapsed_slice_dims=(0,),
                                   start_index_map=(0,)),
        (1,), mode=lax.GatherScatterMode.PROMISE_IN_BOUNDS)


def _edge_common(rows1, rows2, acc0, w2, iota16, g):
    ridx = g * 16 + iota16
    acc = acc0
    e = []
    for j in range(8):
        a = plsc.load_gather(rows1, [ridx, jnp.full((16,), 8 + j, jnp.int32)])
        b = plsc.load_gather(rows2, [ridx, jnp.full((16,), j, jnp.int32)])
        e.append(jnp.exp(jnp.minimum(a + b, 20.0)) + 1.0)
    for q in range(2):
        ea, eb, ec, ed = e[4 * q:4 * q + 4]
        wa, wb, wc, wd = w2[4 * q:4 * q + 4]
        ab = ea * eb
        cd = ec * ed
        n1 = wa * eb + wb * ea
        n2 = wc * ed + wd * ec
        acc = acc - (n1 * cd + n2 * ab) / (ab * cd)
    return ridx, _sigmoid(acc)


def _sc_edge_body(t1, t2, st, en, wp, zr, m1_out, m2_out,
                  sidx, eidx, rows1a, rows2a, rows1b, rows2b, msg, wv,
                  sem1a, sem2a, sem1b, sem2b, m1_sp, m2_sp):
    cid = lax.axis_index("c")
    sid = lax.axis_index("s")
    wid = sid * NC + cid
    rbase = sid * ROWS_PT
    pltpu.sync_copy(zr.at[pl.ds(rbase, ROWS_PT)], m1_sp.at[pl.ds(rbase, ROWS_PT)])
    pltpu.sync_copy(zr.at[pl.ds(rbase, ROWS_PT)], m2_sp.at[pl.ds(rbase, ROWS_PT)])
    pltpu.sync_copy(wp, wv)
    pltpu.sync_copy(st.at[wid], sidx)
    pltpu.sync_copy(en.at[wid], eidx)
    plsc.subcore_barrier()
    iota16 = lax.iota(jnp.int32, 16)
    wv16 = wv[0:16]
    acc0 = _lane(wv16, 8)
    w2 = [_lane(wv16, j) for j in range(8)]

    gbufs = ((rows1a, rows2a, sem1a, sem2a), (rows1b, rows2b, sem1b, sem2b))

    def compute(k, rows1, rows2):
        for g in range(SUB // 16):
            ridx, s = _edge_common(rows1, rows2, acc0, w2, iota16, g)
            for j in range(8):
                p = plsc.load_gather(rows1, [ridx, jnp.full((16,), j, jnp.int32)])
                q = plsc.load_gather(rows2, [ridx, jnp.full((16,), 8 + j, jnp.int32)])
                plsc.store_scatter(msg, [ridx, jnp.full((16,), j, jnp.int32)], s * p)
                plsc.store_scatter(msg, [ridx, jnp.full((16,), 8 + j, jnp.int32)], s * q)
        pltpu.sync_copy(msg, m1_sp.at[eidx.at[k]], add=True)
        pltpu.sync_copy(msg, m2_sp.at[sidx.at[k]], add=True)

    def step(k, b, do_issue):
        r1, r2, s1, s2 = gbufs[b]
        n1, n2, t1s, t2s = gbufs[1 - b]
        pltpu.make_async_copy(t1.at[sidx.at[k]], r1, s1).wait()
        pltpu.make_async_copy(t2.at[eidx.at[k]], r2, s2).wait()
        if do_issue:
            pltpu.async_copy(t1.at[sidx.at[k + 1]], n1, t1s)
            pltpu.async_copy(t2.at[eidx.at[k + 1]], n2, t2s)
        compute(k, r1, r2)

    pltpu.async_copy(t1.at[sidx.at[0]], rows1a, sem1a)
    pltpu.async_copy(t2.at[eidx.at[0]], rows2a, sem2a)

    def pair(i, carry):
        step(2 * i, 0, True)
        step(2 * i + 1, 1, True)
        return carry

    lax.fori_loop(0, (NSUB - 1) // 2, pair, 0)
    step(NSUB - 1, 0, False)
    plsc.subcore_barrier()
    pltpu.sync_copy(m1_sp.at[pl.ds(rbase, ROWS_PT)],
                    m1_out.at[cid, pl.ds(rbase, ROWS_PT)])
    pltpu.sync_copy(m2_sp.at[pl.ds(rbase, ROWS_PT)],
                    m2_out.at[cid, pl.ds(rbase, ROWS_PT)])


_sc_edge = pl.kernel(
    _sc_edge_body,
    out_type=(_f32((NC, NPAD, 16)), _f32((NC, NPAD, 16))),
    mesh=_MESH,
    scratch_types=[
        pltpu.VMEM((NSUB, SUB), jnp.int32),
        pltpu.VMEM((NSUB, SUB), jnp.int32),
        pltpu.VMEM((SUB, 16), jnp.float32),
        pltpu.VMEM((SUB, 16), jnp.float32),
        pltpu.VMEM((SUB, 16), jnp.float32),
        pltpu.VMEM((SUB, 16), jnp.float32),
        pltpu.VMEM((SUB, 16), jnp.float32),
        pltpu.VMEM((32,), jnp.float32),
        pltpu.SemaphoreType.DMA,
        pltpu.SemaphoreType.DMA,
        pltpu.SemaphoreType.DMA,
        pltpu.SemaphoreType.DMA,
        pltpu.VMEM_SHARED((NPAD, 16), jnp.float32),
        pltpu.VMEM_SHARED((NPAD, 16), jnp.float32),
    ],
    compiler_params=pltpu.CompilerParams(needs_layout_passes=False, use_tc_tiling_on_sc=False),
    name="sc_edge_pass",
)


def _sc_final_body(t1, t2, st, en, wp, s_out,
                   sidx, eidx, rows1a, rows2a, rows1b, rows2b, sbuf, wv,
                   sem1a, sem2a, sem1b, sem2b):
    cid = lax.axis_index("c")
    sid = lax.axis_index("s")
    wid = sid * NC + cid
    pltpu.sync_copy(wp, wv)
    pltpu.sync_copy(st.at[wid], sidx)
    pltpu.sync_copy(en.at[wid], eidx)
    iota16 = lax.iota(jnp.int32, 16)
    wv16 = wv[0:16]
    acc0 = _lane(wv16, 8)
    w2 = [_lane(wv16, j) for j in range(8)]

    def compute(k, rows1, rows2):
        for g in range(SUB // 16):
            _, s = _edge_common(rows1, rows2, acc0, w2, iota16, g)
            sbuf[pl.ds(k * SUB + g * 16, 16)] = s

    bufs = ((rows1a, rows2a, sem1a, sem2a), (rows1b, rows2b, sem1b, sem2b))
    pltpu.async_copy(t1.at[sidx.at[0]], rows1a, sem1a)
    pltpu.async_copy(t2.at[eidx.at[0]], rows2a, sem2a)

    def pair(i, carry):
        for b in range(2):
            k = 2 * i + b
            r1, r2, s1, s2 = bufs[b]
            n1, n2, t1s, t2s = bufs[1 - b]
            pltpu.make_async_copy(t1.at[sidx.at[k]], r1, s1).wait()
            pltpu.make_async_copy(t2.at[eidx.at[k]], r2, s2).wait()
            pltpu.async_copy(t1.at[sidx.at[k + 1]], n1, t1s)
            pltpu.async_copy(t2.at[eidx.at[k + 1]], n2, t2s)
            compute(k, r1, r2)
        return carry

    lax.fori_loop(0, (NSUB - 1) // 2, pair, 0)
    k = NSUB - 1
    pltpu.make_async_copy(t1.at[sidx.at[k]], rows1a, sem1a).wait()
    pltpu.make_async_copy(t2.at[eidx.at[k]], rows2a, sem2a).wait()
    compute(k, rows1a, rows2a)
    pltpu.sync_copy(sbuf, s_out.at[wid])


_sc_final = pl.kernel(
    _sc_final_body,
    out_type=_f32((NW, EPW)),
    mesh=_MESH,
    scratch_types=[
        pltpu.VMEM((NSUB, SUB), jnp.int32),
        pltpu.VMEM((NSUB, SUB), jnp.int32),
        pltpu.VMEM((SUB, 16), jnp.float32),
        pltpu.VMEM((SUB, 16), jnp.float32),
        pltpu.VMEM((SUB, 16), jnp.float32),
        pltpu.VMEM((SUB, 16), jnp.float32),
        pltpu.VMEM((EPW,), jnp.float32),
        pltpu.VMEM((32,), jnp.float32),
        pltpu.SemaphoreType.DMA,
        pltpu.SemaphoreType.DMA,
        pltpu.SemaphoreType.DMA,
        pltpu.SemaphoreType.DMA,
    ],
    compiler_params=pltpu.CompilerParams(needs_layout_passes=False, use_tc_tiling_on_sc=False),
    name="sc_edge_final",
)


def _ln(z, g, b, eps=1e-5):
    mu = jnp.mean(z, axis=-1, keepdims=True)
    var = jnp.mean((z - mu) ** 2, axis=-1, keepdims=True)
    return (z - mu) / jnp.sqrt(var + eps) * g + b


def _split_tall(tall, t1_ref, t2_ref, r_ref):
    t1_ref[...] = tall[:, 0:16]
    t2_ref[...] = tall[:, 16:32]
    r_ref[...] = tall[:, 32:40]


def _tc_d0_body(x_ref, w0t_ref, b0_ref, g0_ref, be0_ref, w1t_ref, b1_ref,
                g1_ref, be1_ref, w2t_ref, b2_ref, g2_ref, be2_ref,
                w8_ref, wxt_ref, bias40_ref,
                h_ref, xp_ref, t1_ref, t2_ref, r_ref):
    x = x_ref[...]
    h = jnp.tanh(_ln(x @ w0t_ref[...] + b0_ref[...], g0_ref[...], be0_ref[...]))
    h = jnp.tanh(_ln(h @ w1t_ref[...] + b1_ref[...], g1_ref[...], be1_ref[...]))
    h = jnp.tanh(_ln(h @ w2t_ref[...] + b2_ref[...], g2_ref[...], be2_ref[...]))
    h_ref[...] = h
    xp = x @ wxt_ref[...] + bias40_ref[...]
    xp_ref[...] = xp
    _split_tall((2.0 * h) @ w8_ref[...] + xp, t1_ref, t2_ref, r_ref)


def _tc_d2_body(m1_ref, m2_ref, r_ref, h_ref, xp_ref, w1t_ref, b1_ref, w8_ref,
                hn_ref, t1_ref, t2_ref, rn_ref):
    m = (m1_ref[0, :, 0:8] + m1_ref[1, :, 0:8]
         + m2_ref[0, :, 8:16] + m2_ref[1, :, 8:16])
    t = jnp.tanh(m + r_ref[...])
    hn = jnp.tanh(t @ w1t_ref[...] + b1_ref[...])
    hn_ref[...] = hn
    c8 = h_ref[...] + hn
    _split_tall(c8 @ w8_ref[...] + xp_ref[...], t1_ref, t2_ref, rn_ref)


_BLK = 1000
_NBLK = N_NODES // _BLK


def _full(shape):
    return pl.BlockSpec(shape, lambda i: (0,) * len(shape))


def _rows(w):
    return pl.BlockSpec((_BLK, w), lambda i: (i, 0))


_D0_OUT = (_f32((N_NODES, 8)), _f32((N_NODES, 40)), _f32((N_NODES, 16)),
           _f32((N_NODES, 16)), _f32((N_NODES, 8)))
_tc_d0 = pl.pallas_call(
    _tc_d0_body,
    grid=(_NBLK,),
    in_specs=[_rows(128)] + [_full(s) for s in
              [(128, 8), (1, 8), (1, 8), (1, 8), (8, 8), (1, 8), (1, 8), (1, 8),
               (8, 8), (1, 8), (1, 8), (1, 8), (8, 40), (128, 40), (1, 40)]],
    out_specs=(_rows(8), _rows(40), _rows(16), _rows(16), _rows(8)),
    out_shape=_D0_OUT,
)

_D2_OUT = (_f32((N_NODES, 8)), _f32((N_NODES, 16)), _f32((N_NODES, 16)),
           _f32((N_NODES, 8)))
_tc_d2 = pl.pallas_call(
    _tc_d2_body,
    grid=(_NBLK,),
    in_specs=[pl.BlockSpec((NC, _BLK, 16), lambda i: (0, i, 0)),
              pl.BlockSpec((NC, _BLK, 16), lambda i: (0, i, 0)),
              _rows(8), _rows(8), _rows(40),
              _full((8, 8)), _full((1, 8)), _full((8, 40))],
    out_specs=(_rows(8), _rows(16), _rows(16), _rows(8)),
    out_shape=_D2_OUT,
)


@functools.partial(jax.jit, static_argnums=())
def kernel(nodes, edges, in_W0, in_b0, in_g0, in_be0, in_W1, in_b1, in_g1,
           in_be1, in_W2, in_b2, in_g2, in_be2, e_W0, e_b0, e_W1, e_b1,
           n_W0, n_b0, n_W1, n_b1):
    x = nodes[0]
    ei = edges[0].astype(jnp.int32)
    start_r = ei[:, 0].reshape(NW, NSUB, SUB)
    end_r = ei[:, 1].reshape(NW, NSUB, SUB)

    c = 136
    wbig = jnp.concatenate([n_W0[:, :c], e_W0[:, :c], e_W0[:, c:],
                            n_W0[:, c:2 * c], n_W0[:, 2 * c:]], axis=0).T
    colscale = jnp.concatenate([jnp.ones((8,), jnp.float32),
                                jnp.full((16,), 2.0, jnp.float32),
                                jnp.ones((16,), jnp.float32)])
    wbig = wbig * colscale[None, :]
    w8 = wbig[:8]
    wxt = wbig[8:]
    bias40 = jnp.concatenate([jnp.zeros((8,), jnp.float32), 2.0 * e_b0,
                              jnp.zeros((16,), jnp.float32), n_b0])[None, :]

    wp = jnp.zeros((32,), jnp.float32)
    wp = wp.at[0:8].set(2.0 * e_W1[0]).at[8].set(e_b1[0] + jnp.sum(e_W1[0]))
    zr = jnp.zeros((NPAD, 16), jnp.float32)

    h, xp, t1, t2, r = _tc_d0(
        x, in_W0.T, in_b0[None, :], in_g0[None, :], in_be0[None, :],
        in_W1.T, in_b1[None, :], in_g1[None, :], in_be1[None, :],
        in_W2.T, in_b2[None, :], in_g2[None, :], in_be2[None, :],
        w8, wxt, bias40)

    for _ in range(3):
        m1, m2 = _sc_edge(t1, t2, start_r, end_r, wp, zr)
        h, t1, t2, r = _tc_d2(m1, m2, r, h, xp, n_W1.T, n_b1[None, :], w8)

    s = _sc_final(t1, t2, start_r, end_r, wp)
    return s.reshape(1, N_EDGES)


# speedup vs baseline: 1.2713x; 1.0435x over previous
import functools

import jax
import jax.numpy as jnp
from jax import lax
from jax.experimental import pallas as pl
from jax.experimental.pallas import tpu as pltpu
from jax.experimental.pallas import tpu_sc as plsc

N_NODES = 10000
N_EDGES = 320000
NC = 2
NS = 16
NW = NC * NS
EPW = N_EDGES // NW
SUB = 80
NSUB = EPW // SUB
NPAD = 10240
ROWS_PT = NPAD // NS


def _sigmoid(z):
    return 1.0 / (1.0 + jnp.exp(-z))


def _f32(shape):
    return jax.ShapeDtypeStruct(shape, jnp.float32)


_MESH = plsc.VectorSubcoreMesh(core_axis_name="c", subcore_axis_name="s")


def _lane(v, j):
    return lax.gather(
        v, jnp.full((16, 1), j, jnp.int32),
        lax.GatherDimensionNumbers(offset_dims=(), collapsed_slice_dims=(0,),
                                   start_index_map=(0,)),
        (1,), mode=lax.GatherScatterMode.PROMISE_IN_BOUNDS)


def _edge_common(rows1, rows2, acc0, w2, iota16, g):
    ridx = g * 16 + iota16
    acc = acc0
    e = []
    for j in range(8):
        a = plsc.load_gather(rows1, [ridx, jnp.full((16,), 8 + j, jnp.int32)])
        b = plsc.load_gather(rows2, [ridx, jnp.full((16,), j, jnp.int32)])
        e.append(jnp.exp(jnp.minimum(a + b, 20.0)) + 1.0)
    for q in range(2):
        ea, eb, ec, ed = e[4 * q:4 * q + 4]
        wa, wb, wc, wd = w2[4 * q:4 * q + 4]
        ab = ea * eb
        cd = ec * ed
        n1 = wa * eb + wb * ea
        n2 = wc * ed + wd * ec
        acc = acc - (n1 * cd + n2 * ab) / (ab * cd)
    return ridx, _sigmoid(acc)


def _sc_edge_body(t1, t2, st, en, wp, zr, m1_out, m2_out,
                  sidx, eidx, rows1a, rows2a, rows1b, rows2b, msg, wv,
                  sem1a, sem2a, sem1b, sem2b, m1_sp, m2_sp, t1_sp, t2_sp):
    cid = lax.axis_index("c")
    sid = lax.axis_index("s")
    wid = sid * NC + cid
    rbase = sid * ROWS_PT
    pltpu.sync_copy(t1.at[pl.ds(rbase, ROWS_PT)], t1_sp.at[pl.ds(rbase, ROWS_PT)])
    pltpu.sync_copy(t2.at[pl.ds(rbase, ROWS_PT)], t2_sp.at[pl.ds(rbase, ROWS_PT)])
    pltpu.sync_copy(zr.at[pl.ds(rbase, ROWS_PT)], m1_sp.at[pl.ds(rbase, ROWS_PT)])
    pltpu.sync_copy(zr.at[pl.ds(rbase, ROWS_PT)], m2_sp.at[pl.ds(rbase, ROWS_PT)])
    pltpu.sync_copy(wp, wv)
    pltpu.sync_copy(st.at[wid], sidx)
    pltpu.sync_copy(en.at[wid], eidx)
    plsc.subcore_barrier()
    iota16 = lax.iota(jnp.int32, 16)
    wv16 = wv[0:16]
    acc0 = _lane(wv16, 8)
    w2 = [_lane(wv16, j) for j in range(8)]

    gbufs = ((rows1a, rows2a, sem1a, sem2a), (rows1b, rows2b, sem1b, sem2b))

    def compute(k, rows1, rows2):
        for g in range(SUB // 16):
            ridx, s = _edge_common(rows1, rows2, acc0, w2, iota16, g)
            for j in range(8):
                p = plsc.load_gather(rows1, [ridx, jnp.full((16,), j, jnp.int32)])
                q = plsc.load_gather(rows2, [ridx, jnp.full((16,), 8 + j, jnp.int32)])
                plsc.store_scatter(msg, [ridx, jnp.full((16,), j, jnp.int32)], s * p)
                plsc.store_scatter(msg, [ridx, jnp.full((16,), 8 + j, jnp.int32)], s * q)
        pltpu.sync_copy(msg, m1_sp.at[eidx.at[k]], add=True)
        pltpu.sync_copy(msg, m2_sp.at[sidx.at[k]], add=True)

    def step(k, b, do_issue):
        r1, r2, s1, s2 = gbufs[b]
        n1, n2, t1s, t2s = gbufs[1 - b]
        pltpu.make_async_copy(t1_sp.at[sidx.at[k]], r1, s1).wait()
        pltpu.make_async_copy(t2_sp.at[eidx.at[k]], r2, s2).wait()
        if do_issue:
            pltpu.async_copy(t1_sp.at[sidx.at[k + 1]], n1, t1s)
            pltpu.async_copy(t2_sp.at[eidx.at[k + 1]], n2, t2s)
        compute(k, r1, r2)

    pltpu.async_copy(t1_sp.at[sidx.at[0]], rows1a, sem1a)
    pltpu.async_copy(t2_sp.at[eidx.at[0]], rows2a, sem2a)

    def pair(i, carry):
        step(2 * i, 0, True)
        step(2 * i + 1, 1, True)
        return carry

    lax.fori_loop(0, (NSUB - 1) // 2, pair, 0)
    step(NSUB - 1, 0, False)
    plsc.subcore_barrier()
    pltpu.sync_copy(m1_sp.at[pl.ds(rbase, ROWS_PT)],
                    m1_out.at[cid, pl.ds(rbase, ROWS_PT)])
    pltpu.sync_copy(m2_sp.at[pl.ds(rbase, ROWS_PT)],
                    m2_out.at[cid, pl.ds(rbase, ROWS_PT)])


_sc_edge = pl.kernel(
    _sc_edge_body,
    out_type=(_f32((NC, NPAD, 16)), _f32((NC, NPAD, 16))),
    mesh=_MESH,
    scratch_types=[
        pltpu.VMEM((NSUB, SUB), jnp.int32),
        pltpu.VMEM((NSUB, SUB), jnp.int32),
        pltpu.VMEM((SUB, 16), jnp.float32),
        pltpu.VMEM((SUB, 16), jnp.float32),
        pltpu.VMEM((SUB, 16), jnp.float32),
        pltpu.VMEM((SUB, 16), jnp.float32),
        pltpu.VMEM((SUB, 16), jnp.float32),
        pltpu.VMEM((32,), jnp.float32),
        pltpu.SemaphoreType.DMA,
        pltpu.SemaphoreType.DMA,
        pltpu.SemaphoreType.DMA,
        pltpu.SemaphoreType.DMA,
        pltpu.VMEM_SHARED((NPAD, 16), jnp.float32),
        pltpu.VMEM_SHARED((NPAD, 16), jnp.float32),
        pltpu.VMEM_SHARED((NPAD, 16), jnp.float32),
        pltpu.VMEM_SHARED((NPAD, 16), jnp.float32),
    ],
    compiler_params=pltpu.CompilerParams(needs_layout_passes=False, use_tc_tiling_on_sc=False),
    name="sc_edge_pass",
)


def _sc_final_body(t1, t2, st, en, wp, s_out,
                   sidx, eidx, rows1a, rows2a, rows1b, rows2b, sbuf, wv,
                   sem1a, sem2a, sem1b, sem2b, t1_sp, t2_sp):
    cid = lax.axis_index("c")
    sid = lax.axis_index("s")
    wid = sid * NC + cid
    rbase = sid * ROWS_PT
    pltpu.sync_copy(t1.at[pl.ds(rbase, ROWS_PT)], t1_sp.at[pl.ds(rbase, ROWS_PT)])
    pltpu.sync_copy(t2.at[pl.ds(rbase, ROWS_PT)], t2_sp.at[pl.ds(rbase, ROWS_PT)])
    pltpu.sync_copy(wp, wv)
    pltpu.sync_copy(st.at[wid], sidx)
    pltpu.sync_copy(en.at[wid], eidx)
    plsc.subcore_barrier()
    iota16 = lax.iota(jnp.int32, 16)
    wv16 = wv[0:16]
    acc0 = _lane(wv16, 8)
    w2 = [_lane(wv16, j) for j in range(8)]

    def compute(k, rows1, rows2):
        for g in range(SUB // 16):
            _, s = _edge_common(rows1, rows2, acc0, w2, iota16, g)
            sbuf[pl.ds(k * SUB + g * 16, 16)] = s

    bufs = ((rows1a, rows2a, sem1a, sem2a), (rows1b, rows2b, sem1b, sem2b))
    pltpu.async_copy(t1_sp.at[sidx.at[0]], rows1a, sem1a)
    pltpu.async_copy(t2_sp.at[eidx.at[0]], rows2a, sem2a)

    def pair(i, carry):
        for b in range(2):
            k = 2 * i + b
            r1, r2, s1, s2 = bufs[b]
            n1, n2, t1s, t2s = bufs[1 - b]
            pltpu.make_async_copy(t1_sp.at[sidx.at[k]], r1, s1).wait()
            pltpu.make_async_copy(t2_sp.at[eidx.at[k]], r2, s2).wait()
            pltpu.async_copy(t1_sp.at[sidx.at[k + 1]], n1, t1s)
            pltpu.async_copy(t2_sp.at[eidx.at[k + 1]], n2, t2s)
            compute(k, r1, r2)
        return carry

    lax.fori_loop(0, (NSUB - 1) // 2, pair, 0)
    k = NSUB - 1
    pltpu.make_async_copy(t1_sp.at[sidx.at[k]], rows1a, sem1a).wait()
    pltpu.make_async_copy(t2_sp.at[eidx.at[k]], rows2a, sem2a).wait()
    compute(k, rows1a, rows2a)
    pltpu.sync_copy(sbuf, s_out.at[wid])


_sc_final = pl.kernel(
    _sc_final_body,
    out_type=_f32((NW, EPW)),
    mesh=_MESH,
    scratch_types=[
        pltpu.VMEM((NSUB, SUB), jnp.int32),
        pltpu.VMEM((NSUB, SUB), jnp.int32),
        pltpu.VMEM((SUB, 16), jnp.float32),
        pltpu.VMEM((SUB, 16), jnp.float32),
        pltpu.VMEM((SUB, 16), jnp.float32),
        pltpu.VMEM((SUB, 16), jnp.float32),
        pltpu.VMEM((EPW,), jnp.float32),
        pltpu.VMEM((32,), jnp.float32),
        pltpu.SemaphoreType.DMA,
        pltpu.SemaphoreType.DMA,
        pltpu.SemaphoreType.DMA,
        pltpu.SemaphoreType.DMA,
        pltpu.VMEM_SHARED((NPAD, 16), jnp.float32),
        pltpu.VMEM_SHARED((NPAD, 16), jnp.float32),
    ],
    compiler_params=pltpu.CompilerParams(needs_layout_passes=False, use_tc_tiling_on_sc=False),
    name="sc_edge_final",
)


def _ln(z, g, b, eps=1e-5):
    mu = jnp.mean(z, axis=-1, keepdims=True)
    var = jnp.mean((z - mu) ** 2, axis=-1, keepdims=True)
    return (z - mu) / jnp.sqrt(var + eps) * g + b


def _split_tall(tall, t1_ref, t2_ref, r_ref):
    t1_ref[...] = tall[:, 0:16]
    t2_ref[...] = tall[:, 16:32]
    r_ref[...] = tall[:, 32:40]


def _tc_d0_body(x_ref, w0t_ref, b0_ref, g0_ref, be0_ref, w1t_ref, b1_ref,
                g1_ref, be1_ref, w2t_ref, b2_ref, g2_ref, be2_ref,
                w8_ref, wxt_ref, bias40_ref,
                h_ref, xp_ref, t1_ref, t2_ref, r_ref):
    x = x_ref[...]
    h = jnp.tanh(_ln(x @ w0t_ref[...] + b0_ref[...], g0_ref[...], be0_ref[...]))
    h = jnp.tanh(_ln(h @ w1t_ref[...] + b1_ref[...], g1_ref[...], be1_ref[...]))
    h = jnp.tanh(_ln(h @ w2t_ref[...] + b2_ref[...], g2_ref[...], be2_ref[...]))
    h_ref[...] = h
    xp = x @ wxt_ref[...] + bias40_ref[...]
    xp_ref[...] = xp
    _split_tall((2.0 * h) @ w8_ref[...] + xp, t1_ref, t2_ref, r_ref)


def _tc_d2_body(m1_ref, m2_ref, r_ref, h_ref, xp_ref, w1t_ref, b1_ref, w8_ref,
                hn_ref, t1_ref, t2_ref, rn_ref):
    m = (m1_ref[0, :, 0:8] + m1_ref[1, :, 0:8]
         + m2_ref[0, :, 8:16] + m2_ref[1, :, 8:16])
    t = jnp.tanh(m + r_ref[...])
    hn = jnp.tanh(t @ w1t_ref[...] + b1_ref[...])
    hn_ref[...] = hn
    c8 = h_ref[...] + hn
    _split_tall(c8 @ w8_ref[...] + xp_ref[...], t1_ref, t2_ref, rn_ref)


_BLK = 1000
_NBLK = N_NODES // _BLK


def _full(shape):
    return pl.BlockSpec(shape, lambda i: (0,) * len(shape))


def _rows(w):
    return pl.BlockSpec((_BLK, w), lambda i: (i, 0))


_D0_OUT = (_f32((N_NODES, 8)), _f32((N_NODES, 40)), _f32((N_NODES, 16)),
           _f32((N_NODES, 16)), _f32((N_NODES, 8)))
_tc_d0 = pl.pallas_call(
    _tc_d0_body,
    grid=(_NBLK,),
    in_specs=[_rows(128)] + [_full(s) for s in
              [(128, 8), (1, 8), (1, 8), (1, 8), (8, 8), (1, 8), (1, 8), (1, 8),
               (8, 8), (1, 8), (1, 8), (1, 8), (8, 40), (128, 40), (1, 40)]],
    out_specs=(_rows(8), _rows(40), _rows(16), _rows(16), _rows(8)),
    out_shape=_D0_OUT,
)

_D2_OUT = (_f32((N_NODES, 8)), _f32((N_NODES, 16)), _f32((N_NODES, 16)),
           _f32((N_NODES, 8)))
_tc_d2 = pl.pallas_call(
    _tc_d2_body,
    grid=(_NBLK,),
    in_specs=[pl.BlockSpec((NC, _BLK, 16), lambda i: (0, i, 0)),
              pl.BlockSpec((NC, _BLK, 16), lambda i: (0, i, 0)),
              _rows(8), _rows(8), _rows(40),
              _full((8, 8)), _full((1, 8)), _full((8, 40))],
    out_specs=(_rows(8), _rows(16), _rows(16), _rows(8)),
    out_shape=_D2_OUT,
)


@functools.partial(jax.jit, static_argnums=())
def kernel(nodes, edges, in_W0, in_b0, in_g0, in_be0, in_W1, in_b1, in_g1,
           in_be1, in_W2, in_b2, in_g2, in_be2, e_W0, e_b0, e_W1, e_b1,
           n_W0, n_b0, n_W1, n_b1):
    x = nodes[0]
    ei = edges[0].astype(jnp.int32)
    start_r = ei[:, 0].reshape(NW, NSUB, SUB)
    end_r = ei[:, 1].reshape(NW, NSUB, SUB)

    c = 136
    wbig = jnp.concatenate([n_W0[:, :c], e_W0[:, :c], e_W0[:, c:],
                            n_W0[:, c:2 * c], n_W0[:, 2 * c:]], axis=0).T
    colscale = jnp.concatenate([jnp.ones((8,), jnp.float32),
                                jnp.full((16,), 2.0, jnp.float32),
                                jnp.ones((16,), jnp.float32)])
    wbig = wbig * colscale[None, :]
    w8 = wbig[:8]
    wxt = wbig[8:]
    bias40 = jnp.concatenate([jnp.zeros((8,), jnp.float32), 2.0 * e_b0,
                              jnp.zeros((16,), jnp.float32), n_b0])[None, :]

    wp = jnp.zeros((32,), jnp.float32)
    wp = wp.at[0:8].set(2.0 * e_W1[0]).at[8].set(e_b1[0] + jnp.sum(e_W1[0]))
    zr = jnp.zeros((NPAD, 16), jnp.float32)
    zpad = jnp.zeros((NPAD - N_NODES, 16), jnp.float32)

    h, xp, t1, t2, r = _tc_d0(
        x, in_W0.T, in_b0[None, :], in_g0[None, :], in_be0[None, :],
        in_W1.T, in_b1[None, :], in_g1[None, :], in_be1[None, :],
        in_W2.T, in_b2[None, :], in_g2[None, :], in_be2[None, :],
        w8, wxt, bias40)

    for _ in range(3):
        t1p = jnp.concatenate([t1, zpad], axis=0)
        t2p = jnp.concatenate([t2, zpad], axis=0)
        m1, m2 = _sc_edge(t1p, t2p, start_r, end_r, wp, zr)
        h, t1, t2, r = _tc_d2(m1, m2, r, h, xp, n_W1.T, n_b1[None, :], w8)

    s = _sc_final(jnp.concatenate([t1, zpad], axis=0),
                  jnp.concatenate([t2, zpad], axis=0), start_r, end_r, wp)
    return s.reshape(1, N_EDGES)


# speedup vs baseline: 1.2887x; 1.0137x over previous
import functools

import jax
import jax.numpy as jnp
from jax import lax
from jax.experimental import pallas as pl
from jax.experimental.pallas import tpu as pltpu
from jax.experimental.pallas import tpu_sc as plsc

N_NODES = 10000
N_EDGES = 320000
NC = 2
NS = 16
NW = NC * NS
EPW = N_EDGES // NW
SUB = 80
NSUB = EPW // SUB
NPAD = 10240
ROWS_PT = NPAD // NS


def _sigmoid(z):
    return 1.0 / (1.0 + jnp.exp(-z))


def _f32(shape):
    return jax.ShapeDtypeStruct(shape, jnp.float32)


_MESH = plsc.VectorSubcoreMesh(core_axis_name="c", subcore_axis_name="s")


def _lane(v, j):
    return lax.gather(
        v, jnp.full((16, 1), j, jnp.int32),
        lax.GatherDimensionNumbers(offset_dims=(), collapsed_slice_dims=(0,),
                                   start_index_map=(0,)),
        (1,), mode=lax.GatherScatterMode.PROMISE_IN_BOUNDS)


def _edge_common(rows1, rows2, acc0, w2, iota16, g):
    ridx = g * 16 + iota16
    acc = acc0
    e = []
    for j in range(8):
        a = plsc.load_gather(rows1, [ridx, jnp.full((16,), 8 + j, jnp.int32)])
        b = plsc.load_gather(rows2, [ridx, jnp.full((16,), j, jnp.int32)])
        e.append(jnp.exp(jnp.minimum(a + b, 20.0)) + 1.0)
    for q in range(2):
        ea, eb, ec, ed = e[4 * q:4 * q + 4]
        wa, wb, wc, wd = w2[4 * q:4 * q + 4]
        ab = ea * eb
        cd = ec * ed
        n1 = wa * eb + wb * ea
        n2 = wc * ed + wd * ec
        acc = acc - (n1 * cd + n2 * ab) / (ab * cd)
    return ridx, _sigmoid(acc)


def _sc_edge_body(t1, t2, st, en, wp, zr, m1_out, m2_out,
                  sidx, eidx, rows1a, rows2a, rows1b, rows2b, msg, wv,
                  sem1a, sem2a, sem1b, sem2b, m1_sp, m2_sp, t1_sp, t2_sp):
    cid = lax.axis_index("c")
    sid = lax.axis_index("s")
    wid = sid * NC + cid
    rbase = sid * ROWS_PT
    pltpu.sync_copy(t1.at[pl.ds(rbase, ROWS_PT)], t1_sp.at[pl.ds(rbase, ROWS_PT)])
    pltpu.sync_copy(t2.at[pl.ds(rbase, ROWS_PT)], t2_sp.at[pl.ds(rbase, ROWS_PT)])
    pltpu.sync_copy(zr.at[pl.ds(rbase, ROWS_PT)], m1_sp.at[pl.ds(rbase, ROWS_PT)])
    pltpu.sync_copy(zr.at[pl.ds(rbase, ROWS_PT)], m2_sp.at[pl.ds(rbase, ROWS_PT)])
    pltpu.sync_copy(wp, wv)
    pltpu.sync_copy(st.at[wid], sidx)
    pltpu.sync_copy(en.at[wid], eidx)
    plsc.subcore_barrier()
    iota16 = lax.iota(jnp.int32, 16)
    wv16 = wv[0:16]
    acc0 = _lane(wv16, 8)
    w2 = [_lane(wv16, j) for j in range(8)]

    gbufs = ((rows1a, rows2a, sem1a, sem2a), (rows1b, rows2b, sem1b, sem2b))

    def compute(k, rows1, rows2):
        for g in range(SUB // 16):
            ridx, s = _edge_common(rows1, rows2, acc0, w2, iota16, g)
            for j in range(8):
                p = plsc.load_gather(rows1, [ridx, jnp.full((16,), j, jnp.int32)])
                q = plsc.load_gather(rows2, [ridx, jnp.full((16,), 8 + j, jnp.int32)])
                plsc.store_scatter(msg, [ridx, jnp.full((16,), j, jnp.int32)], s * p)
                plsc.store_scatter(msg, [ridx, jnp.full((16,), 8 + j, jnp.int32)], s * q)
        pltpu.sync_copy(msg, m1_sp.at[eidx.at[k]], add=True)
        pltpu.sync_copy(msg, m2_sp.at[sidx.at[k]], add=True)

    def step(k, b, do_issue):
        r1, r2, s1, s2 = gbufs[b]
        n1, n2, t1s, t2s = gbufs[1 - b]
        pltpu.make_async_copy(t1_sp.at[sidx.at[k]], r1, s1).wait()
        pltpu.make_async_copy(t2_sp.at[eidx.at[k]], r2, s2).wait()
        if do_issue:
            pltpu.async_copy(t1_sp.at[sidx.at[k + 1]], n1, t1s)
            pltpu.async_copy(t2_sp.at[eidx.at[k + 1]], n2, t2s)
        compute(k, r1, r2)

    pltpu.async_copy(t1_sp.at[sidx.at[0]], rows1a, sem1a)
    pltpu.async_copy(t2_sp.at[eidx.at[0]], rows2a, sem2a)

    def pair(i, carry):
        step(2 * i, 0, True)
        step(2 * i + 1, 1, True)
        return carry

    lax.fori_loop(0, (NSUB - 1) // 2, pair, 0)
    step(NSUB - 1, 0, False)
    plsc.subcore_barrier()
    pltpu.sync_copy(m1_sp.at[pl.ds(rbase, ROWS_PT)],
                    m1_out.at[cid, pl.ds(rbase, ROWS_PT)])
    pltpu.sync_copy(m2_sp.at[pl.ds(rbase, ROWS_PT)],
                    m2_out.at[cid, pl.ds(rbase, ROWS_PT)])


_sc_edge = pl.kernel(
    _sc_edge_body,
    out_type=(_f32((NC, NPAD, 16)), _f32((NC, NPAD, 16))),
    mesh=_MESH,
    scratch_types=[
        pltpu.VMEM((NSUB, SUB), jnp.int32),
        pltpu.VMEM((NSUB, SUB), jnp.int32),
        pltpu.VMEM((SUB, 16), jnp.float32),
        pltpu.VMEM((SUB, 16), jnp.float32),
        pltpu.VMEM((SUB, 16), jnp.float32),
        pltpu.VMEM((SUB, 16), jnp.float32),
        pltpu.VMEM((SUB, 16), jnp.float32),
        pltpu.VMEM((32,), jnp.float32),
        pltpu.SemaphoreType.DMA,
        pltpu.SemaphoreType.DMA,
        pltpu.SemaphoreType.DMA,
        pltpu.SemaphoreType.DMA,
        pltpu.VMEM_SHARED((NPAD, 16), jnp.float32),
        pltpu.VMEM_SHARED((NPAD, 16), jnp.float32),
        pltpu.VMEM_SHARED((NPAD, 16), jnp.float32),
        pltpu.VMEM_SHARED((NPAD, 16), jnp.float32),
    ],
    compiler_params=pltpu.CompilerParams(needs_layout_passes=False, use_tc_tiling_on_sc=False),
    name="sc_edge_pass",
)


def _sc_final_body(t1, t2, st, en, wp, s_out,
                   sidx, eidx, rows1a, rows2a, rows1b, rows2b, sbuf, wv,
                   sem1a, sem2a, sem1b, sem2b, t1_sp, t2_sp):
    cid = lax.axis_index("c")
    sid = lax.axis_index("s")
    wid = sid * NC + cid
    rbase = sid * ROWS_PT
    pltpu.sync_copy(t1.at[pl.ds(rbase, ROWS_PT)], t1_sp.at[pl.ds(rbase, ROWS_PT)])
    pltpu.sync_copy(t2.at[pl.ds(rbase, ROWS_PT)], t2_sp.at[pl.ds(rbase, ROWS_PT)])
    pltpu.sync_copy(wp, wv)
    pltpu.sync_copy(st.at[wid], sidx)
    pltpu.sync_copy(en.at[wid], eidx)
    plsc.subcore_barrier()
    iota16 = lax.iota(jnp.int32, 16)
    wv16 = wv[0:16]
    acc0 = _lane(wv16, 8)
    w2 = [_lane(wv16, j) for j in range(8)]

    def compute(k, rows1, rows2):
        for g in range(SUB // 16):
            _, s = _edge_common(rows1, rows2, acc0, w2, iota16, g)
            sbuf[pl.ds(k * SUB + g * 16, 16)] = s

    bufs = ((rows1a, rows2a, sem1a, sem2a), (rows1b, rows2b, sem1b, sem2b))
    pltpu.async_copy(t1_sp.at[sidx.at[0]], rows1a, sem1a)
    pltpu.async_copy(t2_sp.at[eidx.at[0]], rows2a, sem2a)

    def pair(i, carry):
        for b in range(2):
            k = 2 * i + b
            r1, r2, s1, s2 = bufs[b]
            n1, n2, t1s, t2s = bufs[1 - b]
            pltpu.make_async_copy(t1_sp.at[sidx.at[k]], r1, s1).wait()
            pltpu.make_async_copy(t2_sp.at[eidx.at[k]], r2, s2).wait()
            pltpu.async_copy(t1_sp.at[sidx.at[k + 1]], n1, t1s)
            pltpu.async_copy(t2_sp.at[eidx.at[k + 1]], n2, t2s)
            compute(k, r1, r2)
        return carry

    lax.fori_loop(0, (NSUB - 1) // 2, pair, 0)
    k = NSUB - 1
    pltpu.make_async_copy(t1_sp.at[sidx.at[k]], rows1a, sem1a).wait()
    pltpu.make_async_copy(t2_sp.at[eidx.at[k]], rows2a, sem2a).wait()
    compute(k, rows1a, rows2a)
    pltpu.sync_copy(sbuf, s_out.at[wid])


_sc_final = pl.kernel(
    _sc_final_body,
    out_type=_f32((NW, EPW)),
    mesh=_MESH,
    scratch_types=[
        pltpu.VMEM((NSUB, SUB), jnp.int32),
        pltpu.VMEM((NSUB, SUB), jnp.int32),
        pltpu.VMEM((SUB, 16), jnp.float32),
        pltpu.VMEM((SUB, 16), jnp.float32),
        pltpu.VMEM((SUB, 16), jnp.float32),
        pltpu.VMEM((SUB, 16), jnp.float32),
        pltpu.VMEM((EPW,), jnp.float32),
        pltpu.VMEM((32,), jnp.float32),
        pltpu.SemaphoreType.DMA,
        pltpu.SemaphoreType.DMA,
        pltpu.SemaphoreType.DMA,
        pltpu.SemaphoreType.DMA,
        pltpu.VMEM_SHARED((NPAD, 16), jnp.float32),
        pltpu.VMEM_SHARED((NPAD, 16), jnp.float32),
    ],
    compiler_params=pltpu.CompilerParams(needs_layout_passes=False, use_tc_tiling_on_sc=False),
    name="sc_edge_final",
)


def _ln(z, g, b, eps=1e-5):
    mu = jnp.mean(z, axis=-1, keepdims=True)
    var = jnp.mean((z - mu) ** 2, axis=-1, keepdims=True)
    return (z - mu) / jnp.sqrt(var + eps) * g + b


def _split_tall(tall, t1_ref, t2_ref, r_ref):
    t1_ref[...] = tall[:, 0:16]
    t2_ref[...] = tall[:, 16:32]
    r_ref[...] = tall[:, 32:40]


def _tc_d0_body(x_ref, w0t_ref, b0_ref, g0_ref, be0_ref, w1t_ref, b1_ref,
                g1_ref, be1_ref, w2t_ref, b2_ref, g2_ref, be2_ref,
                w8_ref, wxt_ref, bias40_ref,
                h_ref, xp_ref, t1_ref, t2_ref, r_ref):
    x = x_ref[...]
    h = jnp.tanh(_ln(x @ w0t_ref[...] + b0_ref[...], g0_ref[...], be0_ref[...]))
    h = jnp.tanh(_ln(h @ w1t_ref[...] + b1_ref[...], g1_ref[...], be1_ref[...]))
    h = jnp.tanh(_ln(h @ w2t_ref[...] + b2_ref[...], g2_ref[...], be2_ref[...]))
    h_ref[...] = h
    xp = x @ wxt_ref[...] + bias40_ref[...]
    xp_ref[...] = xp
    _split_tall((2.0 * h) @ w8_ref[...] + xp, t1_ref, t2_ref, r_ref)


def _tc_d2_body(m1_ref, m2_ref, r_ref, h_ref, xp_ref, w1t_ref, b1_ref, w8_ref,
                hn_ref, t1_ref, t2_ref, rn_ref):
    m = (m1_ref[0, :, 0:8] + m1_ref[1, :, 0:8]
         + m2_ref[0, :, 8:16] + m2_ref[1, :, 8:16])
    t = jnp.tanh(m + r_ref[...])
    hn = jnp.tanh(t @ w1t_ref[...] + b1_ref[...])
    hn_ref[...] = hn
    c8 = h_ref[...] + hn
    _split_tall(c8 @ w8_ref[...] + xp_ref[...], t1_ref, t2_ref, rn_ref)


_BLK = 1000
_NBLK = N_NODES // _BLK


def _full(shape):
    return pl.BlockSpec(shape, lambda i: (0,) * len(shape))


def _rows(w):
    return pl.BlockSpec((_BLK, w), lambda i: (i, 0))


_D0_OUT = (_f32((N_NODES, 8)), _f32((N_NODES, 40)), _f32((NPAD, 16)),
           _f32((NPAD, 16)), _f32((N_NODES, 8)))
_tc_d0 = pl.pallas_call(
    _tc_d0_body,
    grid=(_NBLK,),
    in_specs=[_rows(128)] + [_full(s) for s in
              [(128, 8), (1, 8), (1, 8), (1, 8), (8, 8), (1, 8), (1, 8), (1, 8),
               (8, 8), (1, 8), (1, 8), (1, 8), (8, 40), (128, 40), (1, 40)]],
    out_specs=(_rows(8), _rows(40), _rows(16), _rows(16), _rows(8)),
    out_shape=_D0_OUT,
)

_D2_OUT = (_f32((N_NODES, 8)), _f32((NPAD, 16)), _f32((NPAD, 16)),
           _f32((N_NODES, 8)))
_tc_d2 = pl.pallas_call(
    _tc_d2_body,
    grid=(_NBLK,),
    in_specs=[pl.BlockSpec((NC, _BLK, 16), lambda i: (0, i, 0)),
              pl.BlockSpec((NC, _BLK, 16), lambda i: (0, i, 0)),
              _rows(8), _rows(8), _rows(40),
              _full((8, 8)), _full((1, 8)), _full((8, 40))],
    out_specs=(_rows(8), _rows(16), _rows(16), _rows(8)),
    out_shape=_D2_OUT,
)


@functools.partial(jax.jit, static_argnums=())
def kernel(nodes, edges, in_W0, in_b0, in_g0, in_be0, in_W1, in_b1, in_g1,
           in_be1, in_W2, in_b2, in_g2, in_be2, e_W0, e_b0, e_W1, e_b1,
           n_W0, n_b0, n_W1, n_b1):
    x = nodes[0]
    ei = edges[0].astype(jnp.int32)
    start_r = ei[:, 0].reshape(NW, NSUB, SUB)
    end_r = ei[:, 1].reshape(NW, NSUB, SUB)

    c = 136
    wbig = jnp.concatenate([n_W0[:, :c], e_W0[:, :c], e_W0[:, c:],
                            n_W0[:, c:2 * c], n_W0[:, 2 * c:]], axis=0).T
    colscale = jnp.concatenate([jnp.ones((8,), jnp.float32),
                                jnp.full((16,), 2.0, jnp.float32),
                                jnp.ones((16,), jnp.float32)])
    wbig = wbig * colscale[None, :]
    w8 = wbig[:8]
    wxt = wbig[8:]
    bias40 = jnp.concatenate([jnp.zeros((8,), jnp.float32), 2.0 * e_b0,
                              jnp.zeros((16,), jnp.float32), n_b0])[None, :]

    wp = jnp.zeros((32,), jnp.float32)
    wp = wp.at[0:8].set(2.0 * e_W1[0]).at[8].set(e_b1[0] + jnp.sum(e_W1[0]))
    zr = jnp.zeros((NPAD, 16), jnp.float32)

    h, xp, t1, t2, r = _tc_d0(
        x, in_W0.T, in_b0[None, :], in_g0[None, :], in_be0[None, :],
        in_W1.T, in_b1[None, :], in_g1[None, :], in_be1[None, :],
        in_W2.T, in_b2[None, :], in_g2[None, :], in_be2[None, :],
        w8, wxt, bias40)

    for _ in range(3):
        m1, m2 = _sc_edge(t1, t2, start_r, end_r, wp, zr)
        h, t1, t2, r = _tc_d2(m1, m2, r, h, xp, n_W1.T, n_b1[None, :], w8)

    s = _sc_final(t1, t2, start_r, end_r, wp)
    return s.reshape(1, N_EDGES)
